# baseline (device time: 839388 ns/iter reference)
import jax
import jax.numpy as jnp
from jax import lax
from jax.experimental import pallas as pl
from jax.experimental.pallas import tpu as pltpu

N_DEV = 4
M_PER = 2048
D = 2048
F_PER = 8192
BF = 512
BM = 1024
N_M = (N_DEV * M_PER) // BM
N_F = F_PER // BF


def _fused_body(x_ref, w1_ref, w2_ref,
                out_ref, p_ref, ring_ref,
                x32_ref, hbuf_ref, acc_ref, stage_ref, comm_ref,
                ag_send, ag_recv, rs_send, rs_recv, credit_sem, load_sem):
    i = lax.axis_index("i")
    left = (i + N_DEV - 1) % N_DEV
    right = (i + 1) % N_DEV
    m = pl.program_id(0)
    f = pl.program_id(1)

    c = jnp.where(m == N_M - 1, 0, (m + 1) // 2)
    half = jnp.where(m == 0, 0, jnp.where(m == N_M - 1, 1, (m + 1) % 2))

    def ring_rdma(h, src_ref):
        return pltpu.make_async_remote_copy(
            src_ref=src_ref,
            dst_ref=ring_ref.at[pl.ds(h * M_PER, M_PER), :],
            send_sem=ag_send.at[h],
            recv_sem=ag_recv.at[h],
            device_id=(right,),
            device_id_type=pl.DeviceIdType.MESH,
        )

    def rs_rdma(s, src_ref, dst_slot):
        return pltpu.make_async_remote_copy(
            src_ref=src_ref,
            dst_ref=comm_ref.at[dst_slot],
            send_sem=rs_send.at[s],
            recv_sem=rs_recv.at[s],
            device_id=(right,),
            device_id_type=pl.DeviceIdType.MESH,
        )

    def local_copy(src, dst):
        cp = pltpu.make_async_copy(src, dst, load_sem)
        cp.start()
        cp.wait()

    def accum_comm(slot, pos):
        for hf in (0, 1):
            local_copy(
                p_ref.at[pl.ds(pos * M_PER + hf * BM, BM), :], stage_ref)
            comm_ref[slot, pl.ds(hf * BM, BM), :] = (
                comm_ref[slot, pl.ds(hf * BM, BM), :] + stage_ref[...])

    @pl.when((m == 0) & (f == 0))
    def _():
        barrier_sem = pltpu.get_barrier_semaphore()
        for nbr in (left, right):
            pl.semaphore_signal(
                barrier_sem, inc=1,
                device_id=(nbr,), device_id_type=pl.DeviceIdType.MESH,
            )
        pl.semaphore_wait(barrier_sem, 2)
        ring_rdma(0, x_ref).start()

    @pl.when((m == 1) & (f == 0))
    def _():
        ring_rdma(0, x_ref).wait_recv()
        ring_rdma(1, ring_ref.at[pl.ds(0 * M_PER, M_PER), :]).start()

    @pl.when((m == 3) & (f == 0))
    def _():
        ring_rdma(1, x_ref).wait_recv()
        ring_rdma(2, ring_ref.at[pl.ds(1 * M_PER, M_PER), :]).start()

    @pl.when((m == 4) & (f == 0))
    def _():
        rs_rdma(0, p_ref.at[pl.ds(1 * M_PER, M_PER), :], 0).start()

    @pl.when((m == 5) & (f == 0))
    def _():
        ring_rdma(2, x_ref).wait_recv()
        rs_rdma(0, x_ref, 0).wait_recv()
        accum_comm(0, 2)
        rs_rdma(1, comm_ref.at[0], 1).start()

    @pl.when((m == N_M - 1) & (f == 0))
    def _():
        rs_rdma(1, x_ref, 1).wait_recv()
        accum_comm(1, 3)
        rs_rdma(1, comm_ref.at[0], 1).wait_send()
        pl.semaphore_signal(
            credit_sem, inc=1,
            device_id=(left,), device_id_type=pl.DeviceIdType.MESH,
        )
        pl.semaphore_wait(credit_sem, 1)
        rs_rdma(2, comm_ref.at[1], 0).start()

    @pl.when(f == 0)
    def _():
        @pl.when(c == 0)
        def _():
            local_copy(x_ref.at[pl.ds(half * BM, BM), :], stage_ref.at[...])

        @pl.when(c > 0)
        def _():
            local_copy(
                ring_ref.at[pl.ds((c - 1) * M_PER + half * BM, BM), :],
                stage_ref.at[...])

        x32_ref[...] = stage_ref[...].astype(jnp.float32)

    @pl.when(f > 0)
    def _():
        hs = hbuf_ref[...]
        hs = hs * jax.nn.sigmoid(hs)
        pp = jnp.dot(hs, w2_ref[...], preferred_element_type=jnp.float32)

        @pl.when(f == 1)
        def _():
            acc_ref[...] = pp

        @pl.when(f > 1)
        def _():
            acc_ref[...] = acc_ref[...] + pp

    @pl.when(f < N_F)
    def _():
        hbuf_ref[...] = jnp.dot(
            x32_ref[...], w1_ref[...], preferred_element_type=jnp.float32)

    @pl.when(f == N_F)
    def _():
        stage_ref[...] = acc_ref[...].astype(jnp.bfloat16)
        local_copy(stage_ref.at[...],
                   p_ref.at[pl.ds(c * M_PER + half * BM, BM), :])

    @pl.when((m == N_M - 1) & (f == N_F))
    def _():
        rs_rdma(2, x_ref, 0).wait_recv()
        for hf in (0, 1):
            local_copy(p_ref.at[pl.ds(0 * M_PER + hf * BM, BM), :],
                       stage_ref)
            comm_ref[0, pl.ds(hf * BM, BM), :] = (
                comm_ref[0, pl.ds(hf * BM, BM), :] + stage_ref[...])
            local_copy(comm_ref.at[0, pl.ds(hf * BM, BM), :],
                       out_ref.at[pl.ds(hf * BM, BM), :])
        rs_rdma(0, p_ref.at[pl.ds(1 * M_PER, M_PER), :], 0).wait_send()
        rs_rdma(2, comm_ref.at[1], 0).wait_send()
        ring_rdma(0, x_ref).wait_send()
        ring_rdma(1, ring_ref.at[pl.ds(0 * M_PER, M_PER), :]).wait_send()
        ring_rdma(2, ring_ref.at[pl.ds(1 * M_PER, M_PER), :]).wait_send()


def _fused_all(x_bf, w1, w2):
    out, _p, _ring = pl.pallas_call(
        _fused_body,
        grid=(N_M, N_F + 1),
        in_specs=[
            pl.BlockSpec(memory_space=pl.ANY),
            pl.BlockSpec((D, BF),
                         lambda m, f: (0, jnp.minimum(f, N_F - 1))),
            pl.BlockSpec((BF, D),
                         lambda m, f: (jnp.maximum(f - 1, 0), 0)),
        ],
        out_specs=[
            pl.BlockSpec(memory_space=pl.ANY),
            pl.BlockSpec(memory_space=pl.ANY),
            pl.BlockSpec(memory_space=pl.ANY),
        ],
        out_shape=[
            jax.ShapeDtypeStruct((M_PER, D), jnp.bfloat16),
            jax.ShapeDtypeStruct((N_DEV * M_PER, D), jnp.bfloat16),
            jax.ShapeDtypeStruct(((N_DEV - 1) * M_PER, D), jnp.bfloat16),
        ],
        scratch_shapes=[
            pltpu.VMEM((BM, D), jnp.float32),
            pltpu.VMEM((BM, BF), jnp.float32),
            pltpu.VMEM((BM, D), jnp.float32),
            pltpu.VMEM((BM, D), jnp.bfloat16),
            pltpu.VMEM((2, M_PER, D), jnp.bfloat16),
            pltpu.SemaphoreType.DMA((N_DEV - 1,)),
            pltpu.SemaphoreType.DMA((N_DEV - 1,)),
            pltpu.SemaphoreType.DMA((N_DEV - 1,)),
            pltpu.SemaphoreType.DMA((N_DEV - 1,)),
            pltpu.SemaphoreType.REGULAR,
            pltpu.SemaphoreType.DMA,
        ],
        compiler_params=pltpu.CompilerParams(
            dimension_semantics=("arbitrary", "arbitrary"),
            collective_id=0,
            vmem_limit_bytes=62 * 1024 * 1024,
        ),
    )(x_bf, w1, w2)
    return out


def kernel(x, W1, W2):
    x_bf = x.astype(jnp.bfloat16)
    return _fused_all(x_bf, W1, W2)


# device time: 813041 ns/iter; 1.0324x vs baseline; 1.0324x over previous
import jax
import jax.numpy as jnp
from jax import lax
from jax.experimental import pallas as pl
from jax.experimental.pallas import tpu as pltpu

N_DEV = 4
M_PER = 2048
D = 2048
F_PER = 8192
BF = 512
BM = 1024
N_M = (N_DEV * M_PER) // BM
N_F = F_PER // BF


def _fused_body(x_ref, w1_ref, w2_ref,
                out_ref, p_ref, ring_ref,
                x32_ref, acc_ref, stage_ref, comm_ref,
                ag_send, ag_recv, rs_send, rs_recv, credit_sem, load_sem):
    i = lax.axis_index("i")
    left = (i + N_DEV - 1) % N_DEV
    right = (i + 1) % N_DEV
    m = pl.program_id(0)
    f = pl.program_id(1)

    c = jnp.where(m == N_M - 1, 0, (m + 1) // 2)
    half = jnp.where(m == 0, 0, jnp.where(m == N_M - 1, 1, (m + 1) % 2))

    def ring_rdma(h, src_ref):
        return pltpu.make_async_remote_copy(
            src_ref=src_ref,
            dst_ref=ring_ref.at[pl.ds(h * M_PER, M_PER), :],
            send_sem=ag_send.at[h],
            recv_sem=ag_recv.at[h],
            device_id=(right,),
            device_id_type=pl.DeviceIdType.MESH,
        )

    def rs_rdma(s, src_ref, dst_slot):
        return pltpu.make_async_remote_copy(
            src_ref=src_ref,
            dst_ref=comm_ref.at[dst_slot],
            send_sem=rs_send.at[s],
            recv_sem=rs_recv.at[s],
            device_id=(right,),
            device_id_type=pl.DeviceIdType.MESH,
        )

    def local_copy(src, dst):
        cp = pltpu.make_async_copy(src, dst, load_sem)
        cp.start()
        cp.wait()

    def accum_comm(slot, pos):
        for hf in (0, 1):
            local_copy(
                p_ref.at[pl.ds(pos * M_PER + hf * BM, BM), :], stage_ref)
            comm_ref[slot, pl.ds(hf * BM, BM), :] = (
                comm_ref[slot, pl.ds(hf * BM, BM), :] + stage_ref[...])

    @pl.when((m == 0) & (f == 0))
    def _():
        barrier_sem = pltpu.get_barrier_semaphore()
        for nbr in (left, right):
            pl.semaphore_signal(
                barrier_sem, inc=1,
                device_id=(nbr,), device_id_type=pl.DeviceIdType.MESH,
            )
        pl.semaphore_wait(barrier_sem, 2)
        ring_rdma(0, x_ref).start()

    @pl.when((m == 1) & (f == 0))
    def _():
        ring_rdma(0, x_ref).wait_recv()
        ring_rdma(1, ring_ref.at[pl.ds(0 * M_PER, M_PER), :]).start()

    @pl.when((m == 3) & (f == 0))
    def _():
        ring_rdma(1, x_ref).wait_recv()
        ring_rdma(2, ring_ref.at[pl.ds(1 * M_PER, M_PER), :]).start()

    @pl.when((m == 4) & (f == 0))
    def _():
        rs_rdma(0, p_ref.at[pl.ds(1 * M_PER, M_PER), :], 0).start()

    @pl.when((m == 5) & (f == 0))
    def _():
        ring_rdma(2, x_ref).wait_recv()
        rs_rdma(0, x_ref, 0).wait_recv()
        accum_comm(0, 2)
        rs_rdma(1, comm_ref.at[0], 1).start()

    @pl.when((m == N_M - 1) & (f == 0))
    def _():
        rs_rdma(1, x_ref, 1).wait_recv()
        accum_comm(1, 3)
        rs_rdma(1, comm_ref.at[0], 1).wait_send()
        pl.semaphore_signal(
            credit_sem, inc=1,
            device_id=(left,), device_id_type=pl.DeviceIdType.MESH,
        )
        pl.semaphore_wait(credit_sem, 1)
        rs_rdma(2, comm_ref.at[1], 0).start()

    @pl.when(f == 0)
    def _():
        @pl.when(c == 0)
        def _():
            local_copy(x_ref.at[pl.ds(half * BM, BM), :], stage_ref.at[...])

        @pl.when(c > 0)
        def _():
            local_copy(
                ring_ref.at[pl.ds((c - 1) * M_PER + half * BM, BM), :],
                stage_ref.at[...])

        x32_ref[...] = stage_ref[...].astype(jnp.float32)

    hh = jnp.dot(x32_ref[...], w1_ref[...], preferred_element_type=jnp.float32)
    hh = hh * jax.nn.sigmoid(hh)
    pp = jnp.dot(hh, w2_ref[...], preferred_element_type=jnp.float32)

    @pl.when(f == 0)
    def _():
        acc_ref[...] = pp

    @pl.when(f > 0)
    def _():
        acc_ref[...] = acc_ref[...] + pp

    @pl.when(f == N_F - 1)
    def _():
        stage_ref[...] = acc_ref[...].astype(jnp.bfloat16)
        local_copy(stage_ref.at[...],
                   p_ref.at[pl.ds(c * M_PER + half * BM, BM), :])

    @pl.when((m == N_M - 1) & (f == N_F - 1))
    def _():
        rs_rdma(2, x_ref, 0).wait_recv()
        for hf in (0, 1):
            local_copy(p_ref.at[pl.ds(0 * M_PER + hf * BM, BM), :],
                       stage_ref)
            comm_ref[0, pl.ds(hf * BM, BM), :] = (
                comm_ref[0, pl.ds(hf * BM, BM), :] + stage_ref[...])
            local_copy(comm_ref.at[0, pl.ds(hf * BM, BM), :],
                       out_ref.at[pl.ds(hf * BM, BM), :])
        rs_rdma(0, p_ref.at[pl.ds(1 * M_PER, M_PER), :], 0).wait_send()
        rs_rdma(2, comm_ref.at[1], 0).wait_send()
        ring_rdma(0, x_ref).wait_send()
        ring_rdma(1, ring_ref.at[pl.ds(0 * M_PER, M_PER), :]).wait_send()
        ring_rdma(2, ring_ref.at[pl.ds(1 * M_PER, M_PER), :]).wait_send()


def _fused_all(x_bf, w1, w2):
    out, _p, _ring = pl.pallas_call(
        _fused_body,
        grid=(N_M, N_F),
        in_specs=[
            pl.BlockSpec(memory_space=pl.ANY),
            pl.BlockSpec((D, BF), lambda m, f: (0, f)),
            pl.BlockSpec((BF, D), lambda m, f: (f, 0)),
        ],
        out_specs=[
            pl.BlockSpec(memory_space=pl.ANY),
            pl.BlockSpec(memory_space=pl.ANY),
            pl.BlockSpec(memory_space=pl.ANY),
        ],
        out_shape=[
            jax.ShapeDtypeStruct((M_PER, D), jnp.bfloat16),
            jax.ShapeDtypeStruct((N_DEV * M_PER, D), jnp.bfloat16),
            jax.ShapeDtypeStruct(((N_DEV - 1) * M_PER, D), jnp.bfloat16),
        ],
        scratch_shapes=[
            pltpu.VMEM((BM, D), jnp.float32),
            pltpu.VMEM((BM, D), jnp.float32),
            pltpu.VMEM((BM, D), jnp.bfloat16),
            pltpu.VMEM((2, M_PER, D), jnp.bfloat16),
            pltpu.SemaphoreType.DMA((N_DEV - 1,)),
            pltpu.SemaphoreType.DMA((N_DEV - 1,)),
            pltpu.SemaphoreType.DMA((N_DEV - 1,)),
            pltpu.SemaphoreType.DMA((N_DEV - 1,)),
            pltpu.SemaphoreType.REGULAR,
            pltpu.SemaphoreType.DMA,
        ],
        compiler_params=pltpu.CompilerParams(
            dimension_semantics=("arbitrary", "arbitrary"),
            collective_id=0,
            vmem_limit_bytes=62 * 1024 * 1024,
        ),
    )(x_bf, w1, w2)
    return out


def kernel(x, W1, W2):
    x_bf = x.astype(jnp.bfloat16)
    return _fused_all(x_bf, W1, W2)
